# Initial kernel scaffold; baseline (speedup 1.0000x reference)
import jax
import jax.numpy as jnp
from jax.experimental import pallas as pl


def kernel(h_v, h_d, h_p, h_t, W_pi, W_M, W_q, W_r, interacts_edge_index, agg_src, agg_dst, last_nodes):
    raise NotImplementedError("write your pallas kernel here")


# speedup vs baseline: 2.5797x; 2.5797x over previous
import functools

import jax
import jax.numpy as jnp
from jax import lax
from jax.experimental import pallas as pl
from jax.experimental.pallas import tpu as pltpu
from jax.experimental.pallas import tpu_sc as plsc

N_ITEM = 10000
N_TARGET = 5000
E1 = 320000
E2 = 160000
DIM = 128
NG = DIM // 16

NC, NS = 2, 16
NW = NC * NS

HALF1 = N_ITEM // NC
HALF2 = N_TARGET // NC
T1 = 5040
T2 = 2560
KO = 40

K1 = 80
EPT1 = E1 // NS
NCH1 = EPT1 // K1

K2 = 80
EPT2 = E2 // NS
NCH2 = EPT2 // K2

KZ = 80
KF = 40

_MESH = plsc.VectorSubcoreMesh(core_axis_name="c", subcore_axis_name="s")

_GATHER_DN = lax.GatherDimensionNumbers(
    offset_dims=(), collapsed_slice_dims=(0,), start_index_map=(0,))


def _lanesum(x):
    lanes = lax.iota(jnp.int32, 16)
    for k in (8, 4, 2, 1):
        idx = jnp.reshape((lanes + k) & 15, (16, 1))
        x = x + lax.gather(x, idx, _GATHER_DN, (1,),
                           mode=lax.GatherScatterMode.PROMISE_IN_BOUNDS)
    return x


def _m8(x):
    return pl.multiple_of(x, 8)

def _route(v, cid, half):
    t = v - cid * half
    t = jnp.minimum(jnp.maximum(t, -1), half)
    return t - (t >> 31) * (half + 1)


def _zero_rows(ref, nrows, width):
    zv = jnp.zeros((16,), jnp.float32)

    def body(i, _):
        for c in range(width // 16):
            ref[i, pl.ds(c * 16, 16)] = zv
        return 0

    lax.fori_loop(0, nrows, body, 0)


def _chunked(sid, nchunks, body):
    def b(j, _):
        body(sid + NS * j)
        return 0

    nj = jnp.where(sid < nchunks % NS, nchunks // NS + 1, nchunks // NS)
    lax.fori_loop(0, nj, b, 0)


@functools.partial(
    pl.kernel,
    out_type=(jax.ShapeDtypeStruct((N_ITEM, DIM), jnp.float32),
              jax.ShapeDtypeStruct((NW, N_ITEM + 16), jnp.float32)),
    mesh=_MESH,
    scratch_types=[
        pltpu.VMEM((K1,), jnp.int32),
        pltpu.VMEM((K1,), jnp.int32),
        pltpu.VMEM((K1,), jnp.int32),
        pltpu.VMEM((K1 + 16,), jnp.int32),
        pltpu.VMEM((K1, DIM), jnp.float32),
        pltpu.VMEM((K1, DIM), jnp.float32),
        pltpu.VMEM((K1, DIM), jnp.float32),
        pltpu.VMEM((K1, DIM), jnp.float32),
        pltpu.VMEM((DIM,), jnp.float32),
        pltpu.VMEM((DIM,), jnp.float32),
        pltpu.VMEM((DIM,), jnp.float32),
        pltpu.VMEM((N_ITEM + 16,), jnp.float32),
        pltpu.VMEM((16,), jnp.float32),
        pltpu.VMEM_SHARED((T1, DIM), jnp.float32),
        pltpu.SemaphoreType.DMA,
        pltpu.SemaphoreType.DMA,
        pltpu.SemaphoreType.DMA,
    ],
)
def _phase1(hv_hbm, hd_hbm, src_hbm, dst_hbm, wpi_hbm, wm1_hbm, wm2_hbm,
            lane0_hbm, out_hbm, dnm_hbm, src_v, dst_v, dst_t, dst_s, s_v,
            d_v, hd_v, st_v, wpi_v, wm1_v, wm2_v, dnm_v, lane0_v, acc_sh,
            sem1, sem2, sem3):
    cid = lax.axis_index("c")
    sid = lax.axis_index("s")
    wid = sid * NC + cid

    _zero_rows(st_v, KZ, DIM)

    def zbody(ch):
        pltpu.sync_copy(st_v, acc_sh.at[pl.ds(_m8(ch * KZ), KZ)])

    _chunked(sid, T1 // KZ, zbody)
    plsc.subcore_barrier()

    zv16 = jnp.zeros((16,), jnp.float32)

    def dzbody(i, _):
        dnm_v[pl.ds(i * 16, 16)] = zv16
        return 0

    lax.fori_loop(0, (N_ITEM + 16) // 16, dzbody, 0)
    pltpu.sync_copy(lane0_hbm, lane0_v)
    gmask = lane0_v[...] * jnp.where(cid == 0, 1.0, 0.0)

    pltpu.sync_copy(wpi_hbm, wpi_v)
    pltpu.sync_copy(wm1_hbm, wm1_v)
    pltpu.sync_copy(wm2_hbm, wm2_v)
    wpis = [wpi_v[pl.ds(c * 16, 16)] for c in range(NG)]
    wm1s = [wm1_v[pl.ds(c * 16, 16)] for c in range(NG)]
    wm2s = [wm2_v[pl.ds(c * 16, 16)] for c in range(NG)]

    base = sid * EPT1

    def chunk_body(i, _):
        off = base + i * K1
        pltpu.sync_copy(src_hbm.at[pl.ds(off, K1)], src_v)
        pltpu.sync_copy(dst_hbm.at[pl.ds(off, K1)], dst_v)
        pltpu.sync_copy(dst_hbm.at[pl.ds(off, K1)], dst_s.at[pl.ds(0, K1)])
        cp1 = pltpu.async_copy(hv_hbm.at[src_v], s_v, sem1)
        cp2 = pltpu.async_copy(hv_hbm.at[dst_v], d_v, sem2)
        cp3 = pltpu.async_copy(hd_hbm.at[pl.ds(_m8(off), K1)], hd_v, sem3)
        for t in range(K1 // 16):
            sl = pl.ds(t * 16, 16)
            dst_t[sl] = _route(dst_s[sl], cid, HALF1)
        cp1.wait()
        cp2.wait()
        cp3.wait()

        def edge_body(e, _):
            accA = jnp.zeros((16,), jnp.float32)
            accB = jnp.zeros((16,), jnp.float32)
            accC = jnp.zeros((16,), jnp.float32)
            svs = []
            for c in range(NG):
                sl = pl.ds(c * 16, 16)
                sv = s_v[e, sl]
                dv = d_v[e, sl]
                hv = hd_v[e, sl]
                p = sv * dv
                accA = accA + p * hv * wpis[c]
                accB = accB + p * wm1s[c]
                accC = accC + hv * wm2s[c]
                svs.append(sv)
            a = _lanesum(accA)
            bc = _lanesum(accB + accC)
            sig = 1.0 / (1.0 + jnp.exp(-bc))
            gv = jnp.exp(a * sig)
            for c in range(NG):
                st_v[e, pl.ds(c * 16, 16)] = svs[c] * gv
            tidx = dst_s[pl.ds(e, 16)][0]
            dnm_v[pl.ds(tidx, 16)] = dnm_v[pl.ds(tidx, 16)] + gv * gmask
            return 0

        lax.fori_loop(0, K1, edge_body, 0)
        pltpu.sync_copy(st_v, acc_sh.at[dst_t], add=True)
        return 0

    lax.fori_loop(0, NCH1, chunk_body, 0)
    plsc.subcore_barrier()

    def obody(ch):
        pltpu.sync_copy(acc_sh.at[pl.ds(_m8(ch * KO), KO)],
                        out_hbm.at[pl.ds(_m8(cid * HALF1 + ch * KO), KO)])

    _chunked(sid, HALF1 // KO, obody)
    pltpu.sync_copy(dnm_v, dnm_hbm.at[wid])


def _dense_ft_body(acc_ref, dnm_ref, wq1t_ref, wr2t_ref, ft_ref, ftq_ref,
                   g2_ref):
    s = acc_ref[...]
    denom = jnp.sum(dnm_ref[...], axis=0) + 1e-12
    ft = s / denom
    ft_ref[...] = ft
    ftq_ref[...] = jnp.dot(ft, wq1t_ref[...],
                           preferred_element_type=jnp.float32)
    g2_ref[...] = jnp.dot(ft, wr2t_ref[...],
                          preferred_element_type=jnp.float32)


_FT_BLK = 400


def _dense_ft(acc, dnm, wq1t, wr2t):
    grid = N_ITEM // _FT_BLK
    return pl.pallas_call(
        _dense_ft_body,
        grid=(grid,),
        in_specs=[
            pl.BlockSpec((_FT_BLK, DIM), lambda i: (i, 0)),
            pl.BlockSpec((NW, _FT_BLK, 1), lambda i: (0, i, 0)),
            pl.BlockSpec((DIM, DIM), lambda i: (0, 0)),
            pl.BlockSpec((DIM, DIM), lambda i: (0, 0)),
        ],
        out_specs=[
            pl.BlockSpec((_FT_BLK, DIM), lambda i: (i, 0)),
            pl.BlockSpec((_FT_BLK, DIM), lambda i: (i, 0)),
            pl.BlockSpec((_FT_BLK, DIM), lambda i: (i, 0)),
        ],
        out_shape=[
            jax.ShapeDtypeStruct((N_ITEM, DIM), jnp.float32),
            jax.ShapeDtypeStruct((N_ITEM, DIM), jnp.float32),
            jax.ShapeDtypeStruct((N_ITEM, DIM), jnp.float32),
        ],
    )(acc, dnm[:, :N_ITEM, None], wq1t, wr2t)


def _stitch_body(p_ref, o_ref):
    o_ref[0:HALF2] = p_ref[0, 0:HALF2]
    o_ref[HALF2:N_TARGET] = p_ref[1, 0:HALF2]


def _stitch(p):
    return pl.pallas_call(
        _stitch_body,
        out_shape=jax.ShapeDtypeStruct((N_TARGET, DIM), jnp.float32),
    )(p)


def _matmul_body(x_ref, wt_ref, o_ref):
    o_ref[...] = jnp.dot(x_ref[...], wt_ref[...],
                         preferred_element_type=jnp.float32)


def _matmul(x, wt, blk):
    n = x.shape[0]
    return pl.pallas_call(
        _matmul_body,
        grid=(n // blk,),
        in_specs=[
            pl.BlockSpec((blk, DIM), lambda i: (i, 0)),
            pl.BlockSpec((DIM, DIM), lambda i: (0, 0)),
        ],
        out_specs=pl.BlockSpec((blk, DIM), lambda i: (i, 0)),
        out_shape=jax.ShapeDtypeStruct((n, DIM), jnp.float32),
    )(x, wt)


@functools.partial(
    pl.kernel,
    out_type=jax.ShapeDtypeStruct((N_TARGET, DIM), jnp.float32),
    mesh=_MESH,
    scratch_types=[
        pltpu.VMEM((KF,), jnp.int32),
        pltpu.VMEM((KF, DIM), jnp.float32),
        pltpu.VMEM((KF, DIM), jnp.float32),
        pltpu.SemaphoreType.DMA,
    ],
)
def _fbuild(htr_hbm, g2_hbm, ln_hbm, f_hbm, idx_v, g_v, h_v, sem):
    cid = lax.axis_index("c")
    sid = lax.axis_index("s")
    wid = sid * NC + cid
    nch = N_TARGET // KF

    def body(j, _):
        off = (wid + NW * j) * KF
        pltpu.sync_copy(ln_hbm.at[pl.ds(off, KF)], idx_v)
        pltpu.async_copy(g2_hbm.at[idx_v], g_v, sem).wait()
        pltpu.sync_copy(htr_hbm.at[pl.ds(_m8(off), KF)], h_v)

        def row(e, _):
            for c in range(NG):
                sl = pl.ds(c * 16, 16)
                h_v[e, sl] = h_v[e, sl] + g_v[e, sl]
            return 0

        lax.fori_loop(0, KF, row, 0)
        pltpu.sync_copy(h_v, f_hbm.at[pl.ds(_m8(off), KF)])
        return 0

    nj = jnp.where(wid < nch % NW, nch // NW + 1, nch // NW)
    lax.fori_loop(0, nj, body, 0)


@functools.partial(
    pl.kernel,
    out_type=jax.ShapeDtypeStruct((NC, T2, DIM), jnp.float32),
    mesh=_MESH,
    scratch_types=[
        pltpu.VMEM((K2,), jnp.int32),
        pltpu.VMEM((K2,), jnp.int32),
        pltpu.VMEM((K2,), jnp.int32),
        pltpu.VMEM((K2 + 16,), jnp.int32),
        pltpu.VMEM((K2, DIM), jnp.float32),
        pltpu.VMEM((K2, DIM), jnp.float32),
        pltpu.VMEM((K2, DIM), jnp.float32),
        pltpu.VMEM((K2, DIM), jnp.float32),
        pltpu.VMEM((K2, DIM), jnp.float32),
        pltpu.VMEM_SHARED((T2, DIM), jnp.float32),
        pltpu.SemaphoreType.DMA,
        pltpu.SemaphoreType.DMA,
        pltpu.SemaphoreType.DMA,
        pltpu.SemaphoreType.DMA,
    ],
)
def _phase2(ftq_hbm, ft_hbm, f_hbm, hpq_hbm, asrc_hbm, adst_hbm,
            out_hbm, si_v, ti_v, ti_t, ti_s, q_v, fv_v, ftv_v, hp_v, st_v,
            acc_sh, sem1, sem2, sem3, sem4):
    cid = lax.axis_index("c")
    sid = lax.axis_index("s")

    _zero_rows(st_v, KZ, DIM)

    def zbody(ch):
        pltpu.sync_copy(st_v, acc_sh.at[pl.ds(_m8(ch * KZ), KZ)])

    _chunked(sid, T2 // KZ, zbody)
    plsc.subcore_barrier()

    base = sid * EPT2

    def chunk_body(i, _):
        off = base + i * K2
        pltpu.sync_copy(asrc_hbm.at[pl.ds(off, K2)], si_v)
        pltpu.sync_copy(adst_hbm.at[pl.ds(off, K2)], ti_v)
        pltpu.sync_copy(adst_hbm.at[pl.ds(off, K2)], ti_s.at[pl.ds(0, K2)])
        cp1 = pltpu.async_copy(ftq_hbm.at[si_v], q_v, sem1)
        cp2 = pltpu.async_copy(f_hbm.at[ti_v], fv_v, sem2)
        cp3 = pltpu.async_copy(ft_hbm.at[si_v], ftv_v, sem3)
        cp4 = pltpu.async_copy(hpq_hbm.at[pl.ds(_m8(off), K2)], hp_v, sem4)
        for t in range(K2 // 16):
            sl = pl.ds(t * 16, 16)
            ti_t[sl] = _route(ti_s[sl], cid, HALF2)
        cp1.wait()
        cp2.wait()
        cp3.wait()
        cp4.wait()

        def edge_body(e, _):
            acc = jnp.zeros((16,), jnp.float32)
            for c in range(NG):
                sl = pl.ds(c * 16, 16)
                x = q_v[e, sl] + hp_v[e, sl]
                e2x = jnp.exp(x + x)
                th = 1.0 - 2.0 / (e2x + 1.0)
                acc = acc + th * fv_v[e, sl]
            cval = _lanesum(acc)
            for c in range(NG):
                sl = pl.ds(c * 16, 16)
                st_v[e, sl] = ftv_v[e, sl] * cval
            return 0

        lax.fori_loop(0, K2, edge_body, 0)
        pltpu.sync_copy(st_v, acc_sh.at[ti_t], add=True)
        return 0

    lax.fori_loop(0, NCH2, chunk_body, 0)
    plsc.subcore_barrier()

    def obody(ch):
        pltpu.sync_copy(acc_sh.at[pl.ds(_m8(ch * KZ), KZ)],
                        out_hbm.at[cid, pl.ds(_m8(ch * KZ), KZ)])

    _chunked(sid, T2 // KZ, obody)


def kernel(h_v, h_d, h_p, h_t, W_pi, W_M, W_q, W_r,
           interacts_edge_index, agg_src, agg_dst, last_nodes):
    src = interacts_edge_index[0]
    dst = interacts_edge_index[1]
    wpi = W_pi.reshape(DIM)
    wm1 = W_M[0, :DIM]
    wm2 = W_M[0, DIM:]
    wq1t = W_q[:, :DIM].T
    wq2t = W_q[:, DIM:].T
    wr1t = W_r[:, :DIM].T
    wr2t = W_r[:, DIM:].T
    lane0 = jnp.zeros((16,), jnp.float32).at[0].set(1.0)

    acc, dnm = _phase1(h_v, h_d, src, dst, wpi, wm1, wm2, lane0)
    ft, ftq, g2 = _dense_ft(acc, dnm, wq1t, wr2t)
    htr = _matmul(h_t, wr1t, 1000)
    hpq = _matmul(h_p, wq2t, 1000)
    f = _fbuild(htr, g2, last_nodes)
    return _stitch(_phase2(ftq, ft, f, hpq, agg_src, agg_dst))


# speedup vs baseline: 3.1233x; 1.2107x over previous
import functools

import jax
import jax.numpy as jnp
from jax import lax
from jax.experimental import pallas as pl
from jax.experimental.pallas import tpu as pltpu
from jax.experimental.pallas import tpu_sc as plsc

N_ITEM = 10000
N_TARGET = 5000
E1 = 320000
E2 = 160000
DIM = 128
NG = DIM // 16

NC, NS = 2, 16
NW = NC * NS

HALF1 = N_ITEM // NC
HALF2 = N_TARGET // NC
T1 = 5040
T2 = 2560
KO = 40

K1 = 80
EPT1 = E1 // NS
NCH1 = EPT1 // K1

K2 = 80
EPT2 = E2 // NS
NCH2 = EPT2 // K2

KZ = 80
KF = 40

_MESH = plsc.VectorSubcoreMesh(core_axis_name="c", subcore_axis_name="s")

_GATHER_DN = lax.GatherDimensionNumbers(
    offset_dims=(), collapsed_slice_dims=(0,), start_index_map=(0,))


def _lanesum(x):
    lanes = lax.iota(jnp.int32, 16)
    for k in (8, 4, 2, 1):
        idx = jnp.reshape((lanes + k) & 15, (16, 1))
        x = x + lax.gather(x, idx, _GATHER_DN, (1,),
                           mode=lax.GatherScatterMode.PROMISE_IN_BOUNDS)
    return x


def _m8(x):
    return pl.multiple_of(x, 8)

def _route(v, cid, half):
    t = v - cid * half
    t = jnp.minimum(jnp.maximum(t, -1), half)
    return t - (t >> 31) * (half + 1)


def _zero_rows(ref, nrows, width):
    zv = jnp.zeros((16,), jnp.float32)

    def body(i, _):
        for c in range(width // 16):
            ref[i, pl.ds(c * 16, 16)] = zv
        return 0

    lax.fori_loop(0, nrows, body, 0)


def _chunked(sid, nchunks, body):
    def b(j, _):
        body(sid + NS * j)
        return 0

    nj = jnp.where(sid < nchunks % NS, nchunks // NS + 1, nchunks // NS)
    lax.fori_loop(0, nj, b, 0)


@functools.partial(
    pl.kernel,
    out_type=(jax.ShapeDtypeStruct((N_ITEM, DIM), jnp.float32),
              jax.ShapeDtypeStruct((NW, N_ITEM + 16), jnp.float32)),
    mesh=_MESH,
    scratch_types=[
        pltpu.VMEM((K1,), jnp.int32),
        pltpu.VMEM((K1,), jnp.int32),
        pltpu.VMEM((K1,), jnp.int32),
        pltpu.VMEM((K1 + 16,), jnp.int32),
        pltpu.VMEM((K1, DIM), jnp.float32),
        pltpu.VMEM((K1, DIM), jnp.float32),
        pltpu.VMEM((K1, DIM), jnp.float32),
        pltpu.VMEM((K1, DIM), jnp.float32),
        pltpu.VMEM((DIM,), jnp.float32),
        pltpu.VMEM((DIM,), jnp.float32),
        pltpu.VMEM((DIM,), jnp.float32),
        pltpu.VMEM((N_ITEM + 16,), jnp.float32),
        pltpu.VMEM((16,), jnp.float32),
        pltpu.VMEM_SHARED((T1, DIM), jnp.float32),
        pltpu.SemaphoreType.DMA,
        pltpu.SemaphoreType.DMA,
        pltpu.SemaphoreType.DMA,
    ],
)
def _phase1(hv_hbm, hd_hbm, src_hbm, dst_hbm, wpi_hbm, wm1_hbm, wm2_hbm,
            lane0_hbm, out_hbm, dnm_hbm, src_v, dst_v, dst_t, dst_s, s_v,
            d_v, hd_v, st_v, wpi_v, wm1_v, wm2_v, dnm_v, lane0_v, acc_sh,
            sem1, sem2, sem3):
    cid = lax.axis_index("c")
    sid = lax.axis_index("s")
    wid = sid * NC + cid

    _zero_rows(st_v, KZ, DIM)

    def zbody(ch):
        pltpu.sync_copy(st_v, acc_sh.at[pl.ds(_m8(ch * KZ), KZ)])

    _chunked(sid, T1 // KZ, zbody)
    plsc.subcore_barrier()

    zv16 = jnp.zeros((16,), jnp.float32)

    def dzbody(i, _):
        dnm_v[pl.ds(i * 16, 16)] = zv16
        return 0

    lax.fori_loop(0, (N_ITEM + 16) // 16, dzbody, 0)
    pltpu.sync_copy(lane0_hbm, lane0_v)
    gmask = lane0_v[...] * jnp.where(cid == 0, 1.0, 0.0)

    pltpu.sync_copy(wpi_hbm, wpi_v)
    pltpu.sync_copy(wm1_hbm, wm1_v)
    pltpu.sync_copy(wm2_hbm, wm2_v)
    wpis = [wpi_v[pl.ds(c * 16, 16)] for c in range(NG)]
    wm1s = [wm1_v[pl.ds(c * 16, 16)] for c in range(NG)]
    wm2s = [wm2_v[pl.ds(c * 16, 16)] for c in range(NG)]

    base = sid * EPT1

    def chunk_body(i, _):
        off = base + i * K1
        pltpu.sync_copy(src_hbm.at[pl.ds(off, K1)], src_v)
        pltpu.sync_copy(dst_hbm.at[pl.ds(off, K1)], dst_v)
        pltpu.sync_copy(dst_hbm.at[pl.ds(off, K1)], dst_s.at[pl.ds(0, K1)])
        cp1 = pltpu.async_copy(hv_hbm.at[src_v], s_v, sem1)
        cp2 = pltpu.async_copy(hv_hbm.at[dst_v], d_v, sem2)
        cp3 = pltpu.async_copy(hd_hbm.at[pl.ds(_m8(off), K1)], hd_v, sem3)
        for t in range(K1 // 16):
            sl = pl.ds(t * 16, 16)
            dst_t[sl] = _route(dst_s[sl], cid, HALF1)
        cp1.wait()
        cp2.wait()
        cp3.wait()

        def edge_body(e4, _):
            gvs = []
            for u in range(4):
                e = e4 * 4 + u
                accA = jnp.zeros((16,), jnp.float32)
                accB = jnp.zeros((16,), jnp.float32)
                accC = jnp.zeros((16,), jnp.float32)
                svs = []
                for c in range(NG):
                    sl = pl.ds(c * 16, 16)
                    sv = s_v[e, sl]
                    dv = d_v[e, sl]
                    hv = hd_v[e, sl]
                    p = sv * dv
                    accA = accA + p * hv * wpis[c]
                    accB = accB + p * wm1s[c]
                    accC = accC + hv * wm2s[c]
                    svs.append(sv)
                a = _lanesum(accA)
                bc = _lanesum(accB + accC)
                sig = 1.0 / (1.0 + jnp.exp(-bc))
                gv = jnp.exp(a * sig)
                for c in range(NG):
                    st_v[e, pl.ds(c * 16, 16)] = svs[c] * gv
                gvs.append(gv)
            for u in range(4):
                e = e4 * 4 + u
                tidx = dst_s[pl.ds(e, 16)][0]
                dnm_v[pl.ds(tidx, 16)] = (dnm_v[pl.ds(tidx, 16)]
                                          + gvs[u] * gmask)
            return 0

        lax.fori_loop(0, K1 // 4, edge_body, 0)
        pltpu.sync_copy(st_v, acc_sh.at[dst_t], add=True)
        return 0

    lax.fori_loop(0, NCH1, chunk_body, 0)
    plsc.subcore_barrier()

    def obody(ch):
        pltpu.sync_copy(acc_sh.at[pl.ds(_m8(ch * KO), KO)],
                        out_hbm.at[pl.ds(_m8(cid * HALF1 + ch * KO), KO)])

    _chunked(sid, HALF1 // KO, obody)
    pltpu.sync_copy(dnm_v, dnm_hbm.at[wid])


def _dense_ft_body(acc_ref, dnm_ref, wq1t_ref, wr2t_ref, ft_ref, ftq_ref,
                   g2_ref):
    s = acc_ref[...]
    denom = jnp.sum(dnm_ref[...], axis=0) + 1e-12
    ft = s / denom
    ft_ref[...] = ft
    ftq_ref[...] = jnp.dot(ft, wq1t_ref[...],
                           preferred_element_type=jnp.float32)
    g2_ref[...] = jnp.dot(ft, wr2t_ref[...],
                          preferred_element_type=jnp.float32)


_FT_BLK = 400


def _dense_ft(acc, dnm, wq1t, wr2t):
    grid = N_ITEM // _FT_BLK
    return pl.pallas_call(
        _dense_ft_body,
        grid=(grid,),
        in_specs=[
            pl.BlockSpec((_FT_BLK, DIM), lambda i: (i, 0)),
            pl.BlockSpec((NW, _FT_BLK, 1), lambda i: (0, i, 0)),
            pl.BlockSpec((DIM, DIM), lambda i: (0, 0)),
            pl.BlockSpec((DIM, DIM), lambda i: (0, 0)),
        ],
        out_specs=[
            pl.BlockSpec((_FT_BLK, DIM), lambda i: (i, 0)),
            pl.BlockSpec((_FT_BLK, DIM), lambda i: (i, 0)),
            pl.BlockSpec((_FT_BLK, DIM), lambda i: (i, 0)),
        ],
        out_shape=[
            jax.ShapeDtypeStruct((N_ITEM, DIM), jnp.float32),
            jax.ShapeDtypeStruct((N_ITEM, DIM), jnp.float32),
            jax.ShapeDtypeStruct((N_ITEM, DIM), jnp.float32),
        ],
    )(acc, dnm[:, :N_ITEM, None], wq1t, wr2t)


def _stitch_body(p_ref, o_ref):
    o_ref[0:HALF2] = p_ref[0, 0:HALF2]
    o_ref[HALF2:N_TARGET] = p_ref[1, 0:HALF2]


def _stitch(p):
    return pl.pallas_call(
        _stitch_body,
        out_shape=jax.ShapeDtypeStruct((N_TARGET, DIM), jnp.float32),
    )(p)


def _matmul_body(x_ref, wt_ref, o_ref):
    o_ref[...] = jnp.dot(x_ref[...], wt_ref[...],
                         preferred_element_type=jnp.float32)


def _matmul(x, wt, blk):
    n = x.shape[0]
    return pl.pallas_call(
        _matmul_body,
        grid=(n // blk,),
        in_specs=[
            pl.BlockSpec((blk, DIM), lambda i: (i, 0)),
            pl.BlockSpec((DIM, DIM), lambda i: (0, 0)),
        ],
        out_specs=pl.BlockSpec((blk, DIM), lambda i: (i, 0)),
        out_shape=jax.ShapeDtypeStruct((n, DIM), jnp.float32),
    )(x, wt)


@functools.partial(
    pl.kernel,
    out_type=jax.ShapeDtypeStruct((N_TARGET, DIM), jnp.float32),
    mesh=_MESH,
    scratch_types=[
        pltpu.VMEM((KF,), jnp.int32),
        pltpu.VMEM((KF, DIM), jnp.float32),
        pltpu.VMEM((KF, DIM), jnp.float32),
        pltpu.SemaphoreType.DMA,
    ],
)
def _fbuild(htr_hbm, g2_hbm, ln_hbm, f_hbm, idx_v, g_v, h_v, sem):
    cid = lax.axis_index("c")
    sid = lax.axis_index("s")
    wid = sid * NC + cid
    nch = N_TARGET // KF

    def body(j, _):
        off = (wid + NW * j) * KF
        pltpu.sync_copy(ln_hbm.at[pl.ds(off, KF)], idx_v)
        pltpu.async_copy(g2_hbm.at[idx_v], g_v, sem).wait()
        pltpu.sync_copy(htr_hbm.at[pl.ds(_m8(off), KF)], h_v)

        def row(e, _):
            for c in range(NG):
                sl = pl.ds(c * 16, 16)
                h_v[e, sl] = h_v[e, sl] + g_v[e, sl]
            return 0

        lax.fori_loop(0, KF, row, 0)
        pltpu.sync_copy(h_v, f_hbm.at[pl.ds(_m8(off), KF)])
        return 0

    nj = jnp.where(wid < nch % NW, nch // NW + 1, nch // NW)
    lax.fori_loop(0, nj, body, 0)


@functools.partial(
    pl.kernel,
    out_type=jax.ShapeDtypeStruct((NC, T2, DIM), jnp.float32),
    mesh=_MESH,
    scratch_types=[
        pltpu.VMEM((K2,), jnp.int32),
        pltpu.VMEM((K2,), jnp.int32),
        pltpu.VMEM((K2,), jnp.int32),
        pltpu.VMEM((K2 + 16,), jnp.int32),
        pltpu.VMEM((K2, DIM), jnp.float32),
        pltpu.VMEM((K2, DIM), jnp.float32),
        pltpu.VMEM((K2, DIM), jnp.float32),
        pltpu.VMEM((K2, DIM), jnp.float32),
        pltpu.VMEM((K2, DIM), jnp.float32),
        pltpu.VMEM_SHARED((T2, DIM), jnp.float32),
        pltpu.SemaphoreType.DMA,
        pltpu.SemaphoreType.DMA,
        pltpu.SemaphoreType.DMA,
        pltpu.SemaphoreType.DMA,
    ],
)
def _phase2(ftq_hbm, ft_hbm, f_hbm, hpq_hbm, asrc_hbm, adst_hbm,
            out_hbm, si_v, ti_v, ti_t, ti_s, q_v, fv_v, ftv_v, hp_v, st_v,
            acc_sh, sem1, sem2, sem3, sem4):
    cid = lax.axis_index("c")
    sid = lax.axis_index("s")

    _zero_rows(st_v, KZ, DIM)

    def zbody(ch):
        pltpu.sync_copy(st_v, acc_sh.at[pl.ds(_m8(ch * KZ), KZ)])

    _chunked(sid, T2 // KZ, zbody)
    plsc.subcore_barrier()

    base = sid * EPT2

    def chunk_body(i, _):
        off = base + i * K2
        pltpu.sync_copy(asrc_hbm.at[pl.ds(off, K2)], si_v)
        pltpu.sync_copy(adst_hbm.at[pl.ds(off, K2)], ti_v)
        pltpu.sync_copy(adst_hbm.at[pl.ds(off, K2)], ti_s.at[pl.ds(0, K2)])
        cp1 = pltpu.async_copy(ftq_hbm.at[si_v], q_v, sem1)
        cp2 = pltpu.async_copy(f_hbm.at[ti_v], fv_v, sem2)
        cp3 = pltpu.async_copy(ft_hbm.at[si_v], ftv_v, sem3)
        cp4 = pltpu.async_copy(hpq_hbm.at[pl.ds(_m8(off), K2)], hp_v, sem4)
        for t in range(K2 // 16):
            sl = pl.ds(t * 16, 16)
            ti_t[sl] = _route(ti_s[sl], cid, HALF2)
        cp1.wait()
        cp2.wait()
        cp3.wait()
        cp4.wait()

        def edge_body(e4, _):
            for u in range(4):
                e = e4 * 4 + u
                acc = jnp.zeros((16,), jnp.float32)
                for c in range(NG):
                    sl = pl.ds(c * 16, 16)
                    x = q_v[e, sl] + hp_v[e, sl]
                    e2x = jnp.exp(x + x)
                    th = 1.0 - 2.0 / (e2x + 1.0)
                    acc = acc + th * fv_v[e, sl]
                cval = _lanesum(acc)
                for c in range(NG):
                    sl = pl.ds(c * 16, 16)
                    st_v[e, sl] = ftv_v[e, sl] * cval
            return 0

        lax.fori_loop(0, K2 // 4, edge_body, 0)
        pltpu.sync_copy(st_v, acc_sh.at[ti_t], add=True)
        return 0

    lax.fori_loop(0, NCH2, chunk_body, 0)
    plsc.subcore_barrier()

    def obody(ch):
        pltpu.sync_copy(acc_sh.at[pl.ds(_m8(ch * KZ), KZ)],
                        out_hbm.at[cid, pl.ds(_m8(ch * KZ), KZ)])

    _chunked(sid, T2 // KZ, obody)


def kernel(h_v, h_d, h_p, h_t, W_pi, W_M, W_q, W_r,
           interacts_edge_index, agg_src, agg_dst, last_nodes):
    src = interacts_edge_index[0]
    dst = interacts_edge_index[1]
    wpi = W_pi.reshape(DIM)
    wm1 = W_M[0, :DIM]
    wm2 = W_M[0, DIM:]
    wq1t = W_q[:, :DIM].T
    wq2t = W_q[:, DIM:].T
    wr1t = W_r[:, :DIM].T
    wr2t = W_r[:, DIM:].T
    lane0 = jnp.zeros((16,), jnp.float32).at[0].set(1.0)

    acc, dnm = _phase1(h_v, h_d, src, dst, wpi, wm1, wm2, lane0)
    ft, ftq, g2 = _dense_ft(acc, dnm, wq1t, wr2t)
    htr = _matmul(h_t, wr1t, 1000)
    hpq = _matmul(h_p, wq2t, 1000)
    f = _fbuild(htr, g2, last_nodes)
    return _stitch(_phase2(ftq, ft, f, hpq, agg_src, agg_dst))
